# baseline (speedup 1.0000x reference)
import functools

import jax
import jax.numpy as jnp
from jax import lax
from jax.experimental import pallas as pl
from jax.experimental.pallas import tpu as pltpu
from jax.experimental.pallas import tpu_sc as plsc

NC = 2
NS = 16
NW = NC * NS

LANES = 16

AW = 1024
ABLOCKS = 976
ATAILP = 144
APW = AW // 4

C = 512
GCHUNK = 128


def _format_table(tbl_t, tail):
    D, V = tbl_t.shape

    mesh = plsc.VectorSubcoreMesh(core_axis_name="c", subcore_axis_name="s")

    @functools.partial(
        pl.kernel,
        mesh=mesh,
        out_type=jax.ShapeDtypeStruct((V // 4, 128), jnp.float32),
        scratch_types=[
            pltpu.VMEM((D, AW), jnp.float32),
            pltpu.VMEM((APW, 128), jnp.float32),
        ],
        compiler_params=pltpu.CompilerParams(needs_layout_passes=False),
    )
    def ka(tbl_hbm, tail_hbm, out_hbm, in_v, out_v):
        wid = lax.axis_index("s") * NC + lax.axis_index("c")

        @pl.when(wid == 0)
        def _():
            pltpu.sync_copy(tail_hbm,
                            out_hbm.at[pl.ds(ABLOCKS * APW, ATAILP)])

        def do_block(c):
            c0 = pl.multiple_of(c * AW, AW)
            pltpu.sync_copy(tbl_hbm.at[:, pl.ds(c0, AW)], in_v)

            def row(r, _):
                base = jnp.full((LANES,), 0, jnp.int32) + (4 * r)
                for l0 in range(0, 128, LANES):
                    q = l0 // 32
                    dbase = l0 % 32
                    rowi = lax.iota(jnp.int32, LANES) + dbase
                    coli = base + q
                    out_v[r, pl.ds(l0, LANES)] = plsc.load_gather(
                        in_v, [rowi, coli])
                return ()

            lax.fori_loop(0, APW, row, (), unroll=False)
            pltpu.sync_copy(out_v,
                            out_hbm.at[pl.ds(pl.multiple_of(c * APW, APW),
                                             APW)])

        def block_loop(i, _):
            c = i * NW + wid

            @pl.when(c < ABLOCKS)
            def _():
                do_block(c)

            return ()

        lax.fori_loop(0, (ABLOCKS + NW - 1) // NW, block_loop, (),
                      unroll=False)

    return ka(tbl_t, tail)


def _gather_tasks(ids_t, tbl2):
    F, S = ids_t.shape
    D = 32
    n_tasks = F * (S // C)
    tasks_per_w = n_tasks // NW
    chunks_per_row = S // C

    mesh = plsc.VectorSubcoreMesh(core_axis_name="c", subcore_axis_name="s")

    @functools.partial(
        pl.kernel,
        mesh=mesh,
        out_type=jax.ShapeDtypeStruct((F, D, S), jnp.float32),
        scratch_types=[
            pltpu.VMEM((C,), jnp.int32),
            pltpu.VMEM((C,), jnp.int32),
            pltpu.VMEM((C, 128), jnp.float32),
            pltpu.VMEM((D, C), jnp.float32),
            pltpu.SemaphoreType.DMA,
        ],
        compiler_params=pltpu.CompilerParams(needs_layout_passes=False),
    )
    def kb(ids_hbm, tbl_hbm, out_hbm, idq_v, off_v, rows_v, out_v, sem):
        wid = lax.axis_index("s") * NC + lax.axis_index("c")

        def task(kk, _):
            t = wid * tasks_per_w + kk
            f = t // chunks_per_row
            s0 = (t % chunks_per_row) * C
            pltpu.sync_copy(ids_hbm.at[f, pl.ds(s0, C)], idq_v)
            for i in range(C // LANES):
                v = idq_v[pl.ds(i * LANES, LANES)]
                off_v[pl.ds(i * LANES, LANES)] = lax.shift_left(
                    lax.bitwise_and(v, 3), 5)
                idq_v[pl.ds(i * LANES, LANES)] = lax.shift_right_logical(v, 2)
            copies = [
                pltpu.async_copy(
                    tbl_hbm.at[idq_v.at[pl.ds(j * GCHUNK, GCHUNK)]],
                    rows_v.at[pl.ds(j * GCHUNK, GCHUNK)],
                    sem,
                )
                for j in range(C // GCHUNK)
            ]
            for cp in copies:
                cp.wait()

            def chunk(j0i, _):
                j0 = j0i * LANES
                rowi = lax.iota(jnp.int32, LANES) + j0
                colb = off_v[pl.ds(j0, LANES)]
                for d in range(D):
                    out_v[d, pl.ds(j0, LANES)] = plsc.load_gather(
                        rows_v, [rowi, colb + d])
                return ()

            lax.fori_loop(0, C // LANES, chunk, (), unroll=False)
            pltpu.sync_copy(out_v, out_hbm.at[f, :, pl.ds(s0, C)])
            return ()

        lax.fori_loop(0, tasks_per_w, task, (), unroll=False)

    return kb(ids_t, tbl2)


def kernel(input_ids, table):
    ids_t = input_ids.T.astype(jnp.int32)
    tail = table[ABLOCKS * AW:].reshape(ATAILP, 128)
    tbl2 = _format_table(table.T, tail)
    out3 = _gather_tasks(ids_t, tbl2)
    return jnp.transpose(out3, (2, 0, 1))
